# baseline (speedup 1.0000x reference)
import functools

import jax
import jax.numpy as jnp
from jax import lax
from jax.experimental import pallas as pl

_B, _H, _L = 16, 512, 64
_HB = 128
_NEG = float("-inf")


def _rc_packed():
    q = lax.broadcasted_iota(jnp.int32, (_L // 2, 2 * _L), 0)
    p = lax.broadcasted_iota(jnp.int32, (_L // 2, 2 * _L), 1)
    r = 2 * q + (p // _L)
    c = p % _L
    return r, c, p % _L


def _pattern(r, c, dtype):
    d = c - r
    pat = (d >= 0) & (d <= 15)
    pat |= (r % 2 == 0) & (d % 2 == 1) & (d >= 17) & (d <= 31)
    pat |= (r % 4 == 0) & (d % 4 == 3) & (d >= 35)
    return pat, pat.astype(dtype)


def _map_kernel(x_ref, out_ref):
    xb = x_ref[0]
    r = lax.broadcasted_iota(jnp.int32, (_L, _L), 0)
    c = lax.broadcasted_iota(jnp.int32, (_L, _L), 1)
    pat, _ = _pattern(r, c, xb.dtype)
    a = jnp.where((c >= r)[None], xb[:, None, :], _NEG)
    s = 1
    while s < _L:
        shifted = jnp.where((c < s)[None], _NEG, jnp.roll(a, s, axis=-1))
        a = jnp.maximum(a, shifted)
        s *= 2
    out_ref[0] = jnp.where(pat[None], a, 0.0)


def _mask_kernel(out_ref):
    r = lax.broadcasted_iota(jnp.int32, (_L, _L), 0)
    c = lax.broadcasted_iota(jnp.int32, (_L, _L), 1)
    _, patf = _pattern(r, c, out_ref.dtype)
    out_ref[...] = jnp.broadcast_to(patf[None, None], out_ref.shape)


@jax.jit
def kernel(x):
    grid = (_B, _H // _HB)
    ori_h = pl.pallas_call(
        _map_kernel,
        grid=grid,
        in_specs=[pl.BlockSpec((1, _HB, _L), lambda b, j: (b, j, 0))],
        out_specs=pl.BlockSpec((1, _HB, _L, _L), lambda b, j: (b, j, 0, 0)),
        out_shape=jax.ShapeDtypeStruct((_B, _H, _L, _L), x.dtype),
    )(x)
    ori_mask = pl.pallas_call(
        _mask_kernel,
        out_shape=jax.ShapeDtypeStruct((_B, 1, _L, _L), x.dtype),
    )()
    return ori_h, ori_mask


# speedup vs baseline: 1.2451x; 1.2451x over previous
import functools

import jax
import jax.numpy as jnp
import numpy as np
from jax import lax
from jax.experimental import pallas as pl
from jax.experimental.pallas import tpu as pltpu
from jax.experimental.pallas import tpu_sc as plsc

_B, _H, _L = 16, 512, 64
_ROWS = _B * _H
_HB = 128
_NLAYER = 32
_CW = _NLAYER * _L
_NW = 32
_RB = 8
_RPW = _ROWS // _NW
_NCHUNK = _RPW // _RB


def _pattern_np():
    r = np.arange(_L)[:, None]
    c = np.arange(_L)[None, :]
    d = c - r
    pat = (d >= 0) & (d <= 15)
    pat |= (r % 2 == 0) & (d % 2 == 1) & (d >= 17) & (d <= 31)
    pat |= (r % 4 == 0) & (d % 4 == 3) & (d >= 35)
    return pat


def _index_tables():
    pat = _pattern_np()
    rr, cc = np.nonzero(pat)
    d = cc - rr
    layer = np.where(
        d <= 15, d,
        np.where(d <= 31, 16 + (d - 17) // 2, 24 + (d - 35) // 4))
    src = layer * _L + rr
    n = src.size
    ik = np.repeat(np.arange(_RB), n).astype(np.int32)
    isrc = np.tile(src, _RB).astype(np.int32)
    ir = np.tile(rr, _RB).astype(np.int32)
    ic = np.tile(cc, _RB).astype(np.int32)
    return ik, isrc, ir, ic, n * _RB


_IK, _ISRC, _IR, _IC, _NIDX = _index_tables()
_NVEC = _NIDX // 16


def _layers_kernel(x_ref, out_ref):
    xb = x_ref[...]
    layers = []
    cur = xb
    layers.append(cur)
    for _ in range(15):
        cur = jnp.maximum(cur, jnp.roll(cur, -1, axis=-1))
        layers.append(cur)
    w16 = cur
    for i in range(8):
        layers.append(jnp.maximum(w16, jnp.roll(w16, -(2 + 2 * i), axis=-1)))
    w32 = jnp.maximum(w16, jnp.roll(w16, -16, axis=-1))
    for i in range(8):
        layers.append(jnp.maximum(w32, jnp.roll(w32, -(4 + 4 * i), axis=-1)))
    out_ref[...] = jnp.concatenate(layers, axis=-1)


def _mask_kernel(out_ref):
    r = lax.broadcasted_iota(jnp.int32, (_L, _L), 0)
    c = lax.broadcasted_iota(jnp.int32, (_L, _L), 1)
    d = c - r
    pat = (d >= 0) & (d <= 15)
    pat |= (r % 2 == 0) & (d % 2 == 1) & (d >= 17) & (d <= 31)
    pat |= (r % 4 == 0) & (d % 4 == 3) & (d >= 35)
    out_ref[...] = jnp.broadcast_to(
        pat.astype(out_ref.dtype)[None, None], out_ref.shape)


def _expand_kernel(c_hbm, zin_hbm, ik_hbm, isrc_hbm, ir_hbm, ic_hbm,
                   out_hbm, cin, vout, ik_v, isrc_v, ir_v, ic_v):
    wid = lax.axis_index("s") * 2 + lax.axis_index("c")
    row0 = wid * _RPW
    b = row0 // _H
    h0 = row0 % _H
    pltpu.sync_copy(ik_hbm, ik_v)
    pltpu.sync_copy(isrc_hbm, isrc_v)
    pltpu.sync_copy(ir_hbm, ir_v)
    pltpu.sync_copy(ic_hbm, ic_v)
    pltpu.sync_copy(zin_hbm, vout)

    def chunk_body(g, _):
        pltpu.sync_copy(c_hbm.at[pl.ds(row0 + g * _RB, _RB)], cin)
        def vec_body(v, _):
            o = v * 16
            ikv = ik_v[pl.ds(o, 16)]
            isv = isrc_v[pl.ds(o, 16)]
            irv = ir_v[pl.ds(o, 16)]
            icv = ic_v[pl.ds(o, 16)]
            vals = plsc.load_gather(cin, [ikv, isv])
            plsc.store_scatter(vout, [ikv, irv, icv], vals)
            return 0

        lax.fori_loop(0, _NVEC, vec_body, 0)
        pltpu.sync_copy(vout, out_hbm.at[b, pl.ds(h0 + g * _RB, _RB)])
        return 0

    lax.fori_loop(0, _NCHUNK, chunk_body, 0)


def _expand(c_tab):
    mesh = plsc.VectorSubcoreMesh(core_axis_name="c", subcore_axis_name="s")
    kern = pl.kernel(
        _expand_kernel,
        out_type=jax.ShapeDtypeStruct((_B, _H, _L, _L), jnp.float32),
        mesh=mesh,
        compiler_params=pltpu.CompilerParams(needs_layout_passes=False),
        scratch_types=[
            pltpu.VMEM((_RB, _CW), jnp.float32),
            pltpu.VMEM((_RB, _L, _L), jnp.float32),
            pltpu.VMEM((_NIDX,), jnp.int32),
            pltpu.VMEM((_NIDX,), jnp.int32),
            pltpu.VMEM((_NIDX,), jnp.int32),
            pltpu.VMEM((_NIDX,), jnp.int32),
        ],
    )
    zin = jnp.zeros((_RB, _L, _L), jnp.float32)
    return kern(c_tab, zin, jnp.asarray(_IK), jnp.asarray(_ISRC),
                jnp.asarray(_IR), jnp.asarray(_IC))


@jax.jit
def kernel(x):
    x2 = x.reshape(_ROWS, _L)
    c_tab = pl.pallas_call(
        _layers_kernel,
        grid=(_ROWS // _HB,),
        in_specs=[pl.BlockSpec((_HB, _L), lambda j: (j, 0))],
        out_specs=pl.BlockSpec((_HB, _CW), lambda j: (j, 0)),
        out_shape=jax.ShapeDtypeStruct((_ROWS, _CW), x.dtype),
    )(x2)
    ori_h = _expand(c_tab)
    ori_mask = pl.pallas_call(
        _mask_kernel,
        out_shape=jax.ShapeDtypeStruct((_B, 1, _L, _L), x.dtype),
    )()
    return ori_h, ori_mask


# speedup vs baseline: 1.6361x; 1.3140x over previous
import functools

import jax
import jax.numpy as jnp
import numpy as np
from jax import lax
from jax.experimental import pallas as pl
from jax.experimental.pallas import tpu as pltpu
from jax.experimental.pallas import tpu_sc as plsc

_B, _H, _L = 16, 512, 64
_ROWS = _B * _H
_HB = 128
_NLAYER = 32
_CW = _NLAYER * _L
_NW = 32
_RB = 8
_RPW = _ROWS // _NW
_NCHUNK = _RPW // _RB


def _pattern_np():
    r = np.arange(_L)[:, None]
    c = np.arange(_L)[None, :]
    d = c - r
    pat = (d >= 0) & (d <= 15)
    pat |= (r % 2 == 0) & (d % 2 == 1) & (d >= 17) & (d <= 31)
    pat |= (r % 4 == 0) & (d % 4 == 3) & (d >= 35)
    return pat


def _index_tables():
    pat = _pattern_np()
    rr, cc = np.nonzero(pat)
    d = cc - rr
    layer = np.where(
        d <= 15, d,
        np.where(d <= 31, 16 + (d - 17) // 2, 24 + (d - 35) // 4))
    src = layer * _L + rr
    n = src.size
    ik = np.repeat(np.arange(_RB), n).astype(np.int32)
    isrc = np.tile(src, _RB).astype(np.int32)
    ir = np.tile(rr, _RB).astype(np.int32)
    ic = np.tile(cc, _RB).astype(np.int32)
    return ik, isrc, ir, ic, n * _RB


_IK, _ISRC, _IR, _IC, _NIDX = _index_tables()
_NVEC = _NIDX // 16


def _layers_kernel(x_ref, out_ref):
    xb = x_ref[...]
    layers = []
    cur = xb
    layers.append(cur)
    for _ in range(15):
        cur = jnp.maximum(cur, jnp.roll(cur, -1, axis=-1))
        layers.append(cur)
    w16 = cur
    for i in range(8):
        layers.append(jnp.maximum(w16, jnp.roll(w16, -(2 + 2 * i), axis=-1)))
    w32 = jnp.maximum(w16, jnp.roll(w16, -16, axis=-1))
    for i in range(8):
        layers.append(jnp.maximum(w32, jnp.roll(w32, -(4 + 4 * i), axis=-1)))
    out_ref[...] = jnp.concatenate(layers, axis=-1)


def _mask_kernel(out_ref):
    r = lax.broadcasted_iota(jnp.int32, (_L, _L), 0)
    c = lax.broadcasted_iota(jnp.int32, (_L, _L), 1)
    d = c - r
    pat = (d >= 0) & (d <= 15)
    pat |= (r % 2 == 0) & (d % 2 == 1) & (d >= 17) & (d <= 31)
    pat |= (r % 4 == 0) & (d % 4 == 3) & (d >= 35)
    out_ref[...] = jnp.broadcast_to(
        pat.astype(out_ref.dtype)[None, None], out_ref.shape)


def _expand_kernel(c_hbm, zin_hbm, ik_hbm, isrc_hbm, ir_hbm, ic_hbm,
                   out_hbm, cin, vout, ik_v, isrc_v, ir_v, ic_v):
    wid = lax.axis_index("s") * 2 + lax.axis_index("c")
    row0 = wid * _RPW
    b = row0 // _H
    h0 = row0 % _H
    pltpu.sync_copy(ik_hbm, ik_v)
    pltpu.sync_copy(isrc_hbm, isrc_v)
    pltpu.sync_copy(ir_hbm, ir_v)
    pltpu.sync_copy(ic_hbm, ic_v)
    pltpu.sync_copy(zin_hbm, vout)

    def chunk_body(g, _):
        pltpu.sync_copy(c_hbm.at[pl.ds(row0 + g * _RB, _RB)], cin)
        @plsc.parallel_loop(0, _NVEC, unroll=8)
        def vec_body(v):
            o = v * 16
            ikv = ik_v[pl.ds(o, 16)]
            isv = isrc_v[pl.ds(o, 16)]
            irv = ir_v[pl.ds(o, 16)]
            icv = ic_v[pl.ds(o, 16)]
            vals = plsc.load_gather(cin, [ikv, isv])
            plsc.store_scatter(vout, [ikv, irv, icv], vals)
        pltpu.sync_copy(vout, out_hbm.at[b, pl.ds(h0 + g * _RB, _RB)])
        return 0

    lax.fori_loop(0, _NCHUNK, chunk_body, 0)


def _expand(c_tab):
    mesh = plsc.VectorSubcoreMesh(core_axis_name="c", subcore_axis_name="s")
    kern = pl.kernel(
        _expand_kernel,
        out_type=jax.ShapeDtypeStruct((_B, _H, _L, _L), jnp.float32),
        mesh=mesh,
        compiler_params=pltpu.CompilerParams(needs_layout_passes=False),
        scratch_types=[
            pltpu.VMEM((_RB, _CW), jnp.float32),
            pltpu.VMEM((_RB, _L, _L), jnp.float32),
            pltpu.VMEM((_NIDX,), jnp.int32),
            pltpu.VMEM((_NIDX,), jnp.int32),
            pltpu.VMEM((_NIDX,), jnp.int32),
            pltpu.VMEM((_NIDX,), jnp.int32),
        ],
    )
    zin = jnp.zeros((_RB, _L, _L), jnp.float32)
    return kern(c_tab, zin, jnp.asarray(_IK), jnp.asarray(_ISRC),
                jnp.asarray(_IR), jnp.asarray(_IC))


@jax.jit
def kernel(x):
    x2 = x.reshape(_ROWS, _L)
    c_tab = pl.pallas_call(
        _layers_kernel,
        grid=(_ROWS // _HB,),
        in_specs=[pl.BlockSpec((_HB, _L), lambda j: (j, 0))],
        out_specs=pl.BlockSpec((_HB, _CW), lambda j: (j, 0)),
        out_shape=jax.ShapeDtypeStruct((_ROWS, _CW), x.dtype),
    )(x2)
    ori_h = _expand(c_tab)
    ori_mask = pl.pallas_call(
        _mask_kernel,
        out_shape=jax.ShapeDtypeStruct((_B, 1, _L, _L), x.dtype),
    )()
    return ori_h, ori_mask


# speedup vs baseline: 2.0419x; 1.2481x over previous
import functools

import jax
import jax.numpy as jnp
import numpy as np
from jax import lax
from jax.experimental import pallas as pl
from jax.experimental.pallas import tpu as pltpu
from jax.experimental.pallas import tpu_sc as plsc

_B, _H, _L = 16, 512, 64
_ROWS = _B * _H
_HB = 128
_NLAYER = 32
_CW = _NLAYER * _L
_NW = 32
_RB = 4
_RPW = _ROWS // _NW
_NCHUNK = _RPW // _RB


def _pattern_np():
    r = np.arange(_L)[:, None]
    c = np.arange(_L)[None, :]
    d = c - r
    pat = (d >= 0) & (d <= 15)
    pat |= (r % 2 == 0) & (d % 2 == 1) & (d >= 17) & (d <= 31)
    pat |= (r % 4 == 0) & (d % 4 == 3) & (d >= 35)
    return pat


def _index_tables():
    pat = _pattern_np()
    rr, cc = np.nonzero(pat)
    d = cc - rr
    layer = np.where(
        d <= 15, d,
        np.where(d <= 31, 16 + (d - 17) // 2, 24 + (d - 35) // 4))
    src = (layer * _L + rr).astype(np.int32)
    return src, rr.astype(np.int32), cc.astype(np.int32), src.size


_ISRC, _IR, _IC, _NIDX = _index_tables()
_NVEC = _NIDX // 16


def _layers_kernel(x_ref, out_ref):
    xb = x_ref[...]
    layers = []
    cur = xb
    layers.append(cur)
    for _ in range(15):
        cur = jnp.maximum(cur, jnp.roll(cur, -1, axis=-1))
        layers.append(cur)
    w16 = cur
    for i in range(8):
        layers.append(jnp.maximum(w16, jnp.roll(w16, -(2 + 2 * i), axis=-1)))
    w32 = jnp.maximum(w16, jnp.roll(w16, -16, axis=-1))
    for i in range(8):
        layers.append(jnp.maximum(w32, jnp.roll(w32, -(4 + 4 * i), axis=-1)))
    out_ref[...] = jnp.concatenate(layers, axis=-1)


def _mask_kernel(out_ref):
    r = lax.broadcasted_iota(jnp.int32, (_L, _L), 0)
    c = lax.broadcasted_iota(jnp.int32, (_L, _L), 1)
    d = c - r
    pat = (d >= 0) & (d <= 15)
    pat |= (r % 2 == 0) & (d % 2 == 1) & (d >= 17) & (d <= 31)
    pat |= (r % 4 == 0) & (d % 4 == 3) & (d >= 35)
    out_ref[...] = jnp.broadcast_to(
        pat.astype(out_ref.dtype)[None, None], out_ref.shape)


def _expand_kernel(c_hbm, zin_hbm, isrc_hbm, ir_hbm, ic_hbm, out_hbm,
                   cin0, cin1, vout0, vout1, isrc_v, ir_v, ic_v,
                   ci_sem0, ci_sem1, vo_sem0, vo_sem1):
    wid = lax.axis_index("s") * 2 + lax.axis_index("c")
    row0 = wid * _RPW
    b = row0 // _H
    h0 = row0 % _H
    cins = (cin0, cin1)
    vouts = (vout0, vout1)
    ci_sems = (ci_sem0, ci_sem1)
    vo_sems = (vo_sem0, vo_sem1)
    pltpu.sync_copy(isrc_hbm, isrc_v)
    pltpu.sync_copy(ir_hbm, ir_v)
    pltpu.sync_copy(ic_hbm, ic_v)
    pltpu.sync_copy(zin_hbm, vout0)
    pltpu.sync_copy(zin_hbm, vout1)

    def cin_dma(ch, t):
        return pltpu.make_async_copy(
            c_hbm.at[pl.ds(row0 + ch * _RB, _RB)], cins[t], ci_sems[t])

    def vout_dma(ch, t):
        return pltpu.make_async_copy(
            vouts[t], out_hbm.at[b, pl.ds(h0 + ch * _RB, _RB)], vo_sems[t])

    cin_dma(0, 0).start()

    def pair_body(g2, _):
        for t in range(2):
            ch = 2 * g2 + t
            cin_dma(ch, t).wait()
            if t == 0:
                cin_dma(ch + 1, 1).start()
            else:
                @pl.when(g2 < _NCHUNK // 2 - 1)
                def _():
                    cin_dma(2 * g2 + 2, 0).start()
            @pl.when(g2 > 0)
            def _():
                vout_dma(ch - 2, t).wait()
            cin_t = cins[t]
            vout_t = vouts[t]
            for k in range(_RB):
                ikv = jnp.full((16,), k, jnp.int32)

                @plsc.parallel_loop(0, _NVEC, unroll=8)
                def vec_body(v):
                    o = v * 16
                    isv = isrc_v[pl.ds(o, 16)]
                    irv = ir_v[pl.ds(o, 16)]
                    icv = ic_v[pl.ds(o, 16)]
                    vals = plsc.load_gather(cin_t, [ikv, isv])
                    plsc.store_scatter(vout_t, [ikv, irv, icv], vals)

            vout_dma(ch, t).start()
        return 0

    lax.fori_loop(0, _NCHUNK // 2, pair_body, 0)
    vout_dma(_NCHUNK - 2, 0).wait()
    vout_dma(_NCHUNK - 1, 1).wait()


def _expand(c_tab):
    mesh = plsc.VectorSubcoreMesh(core_axis_name="c", subcore_axis_name="s")
    kern = pl.kernel(
        _expand_kernel,
        out_type=jax.ShapeDtypeStruct((_B, _H, _L, _L), jnp.float32),
        mesh=mesh,
        compiler_params=pltpu.CompilerParams(needs_layout_passes=False),
        scratch_types=[
            pltpu.VMEM((_RB, _CW), jnp.float32),
            pltpu.VMEM((_RB, _CW), jnp.float32),
            pltpu.VMEM((_RB, _L, _L), jnp.float32),
            pltpu.VMEM((_RB, _L, _L), jnp.float32),
            pltpu.VMEM((_NIDX,), jnp.int32),
            pltpu.VMEM((_NIDX,), jnp.int32),
            pltpu.VMEM((_NIDX,), jnp.int32),
            pltpu.SemaphoreType.DMA,
            pltpu.SemaphoreType.DMA,
            pltpu.SemaphoreType.DMA,
            pltpu.SemaphoreType.DMA,
        ],
    )
    zin = jnp.zeros((_RB, _L, _L), jnp.float32)
    return kern(c_tab, zin, jnp.asarray(_ISRC),
                jnp.asarray(_IR), jnp.asarray(_IC))


@jax.jit
def kernel(x):
    x2 = x.reshape(_ROWS, _L)
    c_tab = pl.pallas_call(
        _layers_kernel,
        grid=(_ROWS // _HB,),
        in_specs=[pl.BlockSpec((_HB, _L), lambda j: (j, 0))],
        out_specs=pl.BlockSpec((_HB, _CW), lambda j: (j, 0)),
        out_shape=jax.ShapeDtypeStruct((_ROWS, _CW), x.dtype),
    )(x2)
    ori_h = _expand(c_tab)
    ori_mask = pl.pallas_call(
        _mask_kernel,
        out_shape=jax.ShapeDtypeStruct((_B, 1, _L, _L), x.dtype),
    )()
    return ori_h, ori_mask


# speedup vs baseline: 2.0437x; 1.0009x over previous
import functools

import jax
import jax.numpy as jnp
import numpy as np
from jax import lax
from jax.experimental import pallas as pl
from jax.experimental.pallas import tpu as pltpu
from jax.experimental.pallas import tpu_sc as plsc

_B, _H, _L = 16, 512, 64
_ROWS = _B * _H
_HB = 128
_NLAYER = 32
_CW = _NLAYER * _L
_NW = 32
_RB = 4
_RPW = _ROWS // _NW
_NCHUNK = _RPW // _RB


def _pattern_np():
    r = np.arange(_L)[:, None]
    c = np.arange(_L)[None, :]
    d = c - r
    pat = (d >= 0) & (d <= 15)
    pat |= (r % 2 == 0) & (d % 2 == 1) & (d >= 17) & (d <= 31)
    pat |= (r % 4 == 0) & (d % 4 == 3) & (d >= 35)
    return pat


def _index_tables():
    pat = _pattern_np()
    rr, cc = np.nonzero(pat)
    d = cc - rr
    layer = np.where(
        d <= 15, d,
        np.where(d <= 31, 16 + (d - 17) // 2, 24 + (d - 35) // 4))
    src = (layer * _L + rr).astype(np.int32)
    return src, rr.astype(np.int32), cc.astype(np.int32), src.size


_ISRC, _IR, _IC, _NIDX = _index_tables()
_NVEC = _NIDX // 16


def _layers_kernel(x_ref, out_ref):
    xb = x_ref[...]
    layers = []
    cur = xb
    layers.append(cur)
    for _ in range(15):
        cur = jnp.maximum(cur, jnp.roll(cur, -1, axis=-1))
        layers.append(cur)
    w16 = cur
    for i in range(8):
        layers.append(jnp.maximum(w16, jnp.roll(w16, -(2 + 2 * i), axis=-1)))
    w32 = jnp.maximum(w16, jnp.roll(w16, -16, axis=-1))
    for i in range(8):
        layers.append(jnp.maximum(w32, jnp.roll(w32, -(4 + 4 * i), axis=-1)))
    out_ref[...] = jnp.concatenate(layers, axis=-1)


def _mask_kernel(out_ref):
    r = lax.broadcasted_iota(jnp.int32, (_L, _L), 0)
    c = lax.broadcasted_iota(jnp.int32, (_L, _L), 1)
    d = c - r
    pat = (d >= 0) & (d <= 15)
    pat |= (r % 2 == 0) & (d % 2 == 1) & (d >= 17) & (d <= 31)
    pat |= (r % 4 == 0) & (d % 4 == 3) & (d >= 35)
    out_ref[...] = jnp.broadcast_to(
        pat.astype(out_ref.dtype)[None, None], out_ref.shape)


def _expand_kernel(c_hbm, zin_hbm, isrc_hbm, ir_hbm, ic_hbm, out_hbm,
                   cin0, cin1, vout0, vout1, isrc_v, ir_v, ic_v,
                   ci_sem0, ci_sem1, vo_sem0, vo_sem1):
    wid = lax.axis_index("s") * 2 + lax.axis_index("c")
    row0 = wid * _RPW
    b = row0 // _H
    h0 = row0 % _H
    cins = (cin0, cin1)
    vouts = (vout0, vout1)
    ci_sems = (ci_sem0, ci_sem1)
    vo_sems = (vo_sem0, vo_sem1)
    pltpu.sync_copy(isrc_hbm, isrc_v)
    pltpu.sync_copy(ir_hbm, ir_v)
    pltpu.sync_copy(ic_hbm, ic_v)
    pltpu.sync_copy(zin_hbm, vout0)
    pltpu.sync_copy(zin_hbm, vout1)

    def cin_dma(ch, t):
        return pltpu.make_async_copy(
            c_hbm.at[pl.ds(row0 + ch * _RB, _RB)], cins[t], ci_sems[t])

    def vout_dma(ch, t):
        return pltpu.make_async_copy(
            vouts[t], out_hbm.at[b, pl.ds(h0 + ch * _RB, _RB)], vo_sems[t])

    cin_dma(0, 0).start()

    def pair_body(g2, _):
        for t in range(2):
            ch = 2 * g2 + t
            cin_dma(ch, t).wait()
            if t == 0:
                cin_dma(ch + 1, 1).start()
            else:
                @pl.when(g2 < _NCHUNK // 2 - 1)
                def _():
                    cin_dma(2 * g2 + 2, 0).start()
            @pl.when(g2 > 0)
            def _():
                vout_dma(ch - 2, t).wait()
            cin_t = cins[t]
            vout_t = vouts[t]
            for k in range(_RB):
                ikv = jnp.full((16,), k, jnp.int32)

                @plsc.parallel_loop(0, _NVEC, unroll=23)
                def vec_body(v):
                    o = v * 16
                    isv = isrc_v[pl.ds(o, 16)]
                    irv = ir_v[pl.ds(o, 16)]
                    icv = ic_v[pl.ds(o, 16)]
                    vals = plsc.load_gather(cin_t, [ikv, isv])
                    plsc.store_scatter(vout_t, [ikv, irv, icv], vals)

            vout_dma(ch, t).start()
        return 0

    lax.fori_loop(0, _NCHUNK // 2, pair_body, 0)
    vout_dma(_NCHUNK - 2, 0).wait()
    vout_dma(_NCHUNK - 1, 1).wait()


def _expand(c_tab):
    mesh = plsc.VectorSubcoreMesh(core_axis_name="c", subcore_axis_name="s")
    kern = pl.kernel(
        _expand_kernel,
        out_type=jax.ShapeDtypeStruct((_B, _H, _L, _L), jnp.float32),
        mesh=mesh,
        compiler_params=pltpu.CompilerParams(needs_layout_passes=False),
        scratch_types=[
            pltpu.VMEM((_RB, _CW), jnp.float32),
            pltpu.VMEM((_RB, _CW), jnp.float32),
            pltpu.VMEM((_RB, _L, _L), jnp.float32),
            pltpu.VMEM((_RB, _L, _L), jnp.float32),
            pltpu.VMEM((_NIDX,), jnp.int32),
            pltpu.VMEM((_NIDX,), jnp.int32),
            pltpu.VMEM((_NIDX,), jnp.int32),
            pltpu.SemaphoreType.DMA,
            pltpu.SemaphoreType.DMA,
            pltpu.SemaphoreType.DMA,
            pltpu.SemaphoreType.DMA,
        ],
    )
    zin = jnp.zeros((_RB, _L, _L), jnp.float32)
    return kern(c_tab, zin, jnp.asarray(_ISRC),
                jnp.asarray(_IR), jnp.asarray(_IC))


@jax.jit
def kernel(x):
    x2 = x.reshape(_ROWS, _L)
    c_tab = pl.pallas_call(
        _layers_kernel,
        grid=(_ROWS // _HB,),
        in_specs=[pl.BlockSpec((_HB, _L), lambda j: (j, 0))],
        out_specs=pl.BlockSpec((_HB, _CW), lambda j: (j, 0)),
        out_shape=jax.ShapeDtypeStruct((_ROWS, _CW), x.dtype),
    )(x2)
    ori_h = _expand(c_tab)
    ori_mask = pl.pallas_call(
        _mask_kernel,
        out_shape=jax.ShapeDtypeStruct((_B, 1, _L, _L), x.dtype),
    )()
    return ori_h, ori_mask


# speedup vs baseline: 2.1190x; 1.0368x over previous
import functools

import jax
import jax.numpy as jnp
import numpy as np
from jax import lax
from jax.experimental import pallas as pl
from jax.experimental.pallas import tpu as pltpu
from jax.experimental.pallas import tpu_sc as plsc

_B, _H, _L = 16, 512, 64
_ROWS = _B * _H
_HB = 128
_NLAYER = 32
_CW = _NLAYER * _L
_NW = 32
_RB = 4
_RPW = _ROWS // _NW
_NCHUNK = _RPW // _RB


def _pattern_np():
    r = np.arange(_L)[:, None]
    c = np.arange(_L)[None, :]
    d = c - r
    pat = (d >= 0) & (d <= 15)
    pat |= (r % 2 == 0) & (d % 2 == 1) & (d >= 17) & (d <= 31)
    pat |= (r % 4 == 0) & (d % 4 == 3) & (d >= 35)
    return pat


def _index_tables():
    pat = _pattern_np()
    rr, cc = np.nonzero(pat)
    d = cc - rr
    layer = np.where(
        d <= 15, d,
        np.where(d <= 31, 16 + (d - 17) // 2, 24 + (d - 35) // 4))
    src = (layer * _L + rr).astype(np.int32)
    return src, rr.astype(np.int32), cc.astype(np.int32), src.size


_ISRC, _IR, _IC, _NIDX = _index_tables()
_NVEC = _NIDX // 16


def _layers_kernel(x_ref, out_ref):
    xb = x_ref[...]
    layers = []
    cur = xb
    layers.append(cur)
    for _ in range(15):
        cur = jnp.maximum(cur, jnp.roll(cur, -1, axis=-1))
        layers.append(cur)
    w16 = cur
    for i in range(8):
        layers.append(jnp.maximum(w16, jnp.roll(w16, -(2 + 2 * i), axis=-1)))
    w32 = jnp.maximum(w16, jnp.roll(w16, -16, axis=-1))
    for i in range(8):
        layers.append(jnp.maximum(w32, jnp.roll(w32, -(4 + 4 * i), axis=-1)))
    out_ref[...] = jnp.concatenate(layers, axis=-1)


def _mask_kernel(out_ref):
    r = lax.broadcasted_iota(jnp.int32, (_L, _L), 0)
    c = lax.broadcasted_iota(jnp.int32, (_L, _L), 1)
    d = c - r
    pat = (d >= 0) & (d <= 15)
    pat |= (r % 2 == 0) & (d % 2 == 1) & (d >= 17) & (d <= 31)
    pat |= (r % 4 == 0) & (d % 4 == 3) & (d >= 35)
    out_ref[...] = jnp.broadcast_to(
        pat.astype(out_ref.dtype)[None, None], out_ref.shape)


def _expand_kernel(c_hbm, zin_hbm, isrc_hbm, ir_hbm, ic_hbm, out_hbm,
                   cin0, cin1, vout0, vout1, isrc_v, ir_v, ic_v,
                   ci_sem0, ci_sem1, vo_sem0, vo_sem1):
    wid = lax.axis_index("s") * 2 + lax.axis_index("c")
    row0 = wid * _RPW
    b = row0 // _H
    h0 = row0 % _H
    cins = (cin0, cin1)
    vouts = (vout0, vout1)
    ci_sems = (ci_sem0, ci_sem1)
    vo_sems = (vo_sem0, vo_sem1)
    pltpu.sync_copy(isrc_hbm, isrc_v)
    pltpu.sync_copy(ir_hbm, ir_v)
    pltpu.sync_copy(ic_hbm, ic_v)
    pltpu.sync_copy(zin_hbm, vout0)
    pltpu.sync_copy(zin_hbm, vout1)

    def cin_dma(ch, t):
        return pltpu.make_async_copy(
            c_hbm.at[pl.ds(row0 + ch * _RB, _RB)], cins[t], ci_sems[t])

    def vout_dma(ch, t):
        return pltpu.make_async_copy(
            vouts[t], out_hbm.at[b, pl.ds(h0 + ch * _RB, _RB)], vo_sems[t])

    cin_dma(0, 0).start()

    def pair_body(g2, _):
        for t in range(2):
            ch = 2 * g2 + t
            cin_dma(ch, t).wait()
            if t == 0:
                cin_dma(ch + 1, 1).start()
            else:
                @pl.when(g2 < _NCHUNK // 2 - 1)
                def _():
                    cin_dma(2 * g2 + 2, 0).start()
            @pl.when(g2 > 0)
            def _():
                vout_dma(ch - 2, t).wait()
            cin_t = cins[t]
            vout_t = vouts[t]

            @plsc.parallel_loop(0, _NVEC, unroll=8)
            def vec_body(v):
                o = v * 16
                isv = isrc_v[pl.ds(o, 16)]
                irv = ir_v[pl.ds(o, 16)]
                icv = ic_v[pl.ds(o, 16)]
                for k in range(_RB):
                    ikv = jnp.full((16,), k, jnp.int32)
                    vals = plsc.load_gather(cin_t, [ikv, isv])
                    plsc.store_scatter(vout_t, [ikv, irv, icv], vals)

            vout_dma(ch, t).start()
        return 0

    lax.fori_loop(0, _NCHUNK // 2, pair_body, 0)
    vout_dma(_NCHUNK - 2, 0).wait()
    vout_dma(_NCHUNK - 1, 1).wait()


def _expand(c_tab):
    mesh = plsc.VectorSubcoreMesh(core_axis_name="c", subcore_axis_name="s")
    kern = pl.kernel(
        _expand_kernel,
        out_type=jax.ShapeDtypeStruct((_B, _H, _L, _L), jnp.float32),
        mesh=mesh,
        compiler_params=pltpu.CompilerParams(needs_layout_passes=False),
        scratch_types=[
            pltpu.VMEM((_RB, _CW), jnp.float32),
            pltpu.VMEM((_RB, _CW), jnp.float32),
            pltpu.VMEM((_RB, _L, _L), jnp.float32),
            pltpu.VMEM((_RB, _L, _L), jnp.float32),
            pltpu.VMEM((_NIDX,), jnp.int32),
            pltpu.VMEM((_NIDX,), jnp.int32),
            pltpu.VMEM((_NIDX,), jnp.int32),
            pltpu.SemaphoreType.DMA,
            pltpu.SemaphoreType.DMA,
            pltpu.SemaphoreType.DMA,
            pltpu.SemaphoreType.DMA,
        ],
    )
    zin = jnp.zeros((_RB, _L, _L), jnp.float32)
    return kern(c_tab, zin, jnp.asarray(_ISRC),
                jnp.asarray(_IR), jnp.asarray(_IC))


@jax.jit
def kernel(x):
    x2 = x.reshape(_ROWS, _L)
    c_tab = pl.pallas_call(
        _layers_kernel,
        grid=(_ROWS // _HB,),
        in_specs=[pl.BlockSpec((_HB, _L), lambda j: (j, 0))],
        out_specs=pl.BlockSpec((_HB, _CW), lambda j: (j, 0)),
        out_shape=jax.ShapeDtypeStruct((_ROWS, _CW), x.dtype),
    )(x2)
    ori_h = _expand(c_tab)
    ori_mask = pl.pallas_call(
        _mask_kernel,
        out_shape=jax.ShapeDtypeStruct((_B, 1, _L, _L), x.dtype),
    )()
    return ori_h, ori_mask


# speedup vs baseline: 2.2674x; 1.0701x over previous
import functools

import jax
import jax.numpy as jnp
import numpy as np
from jax import lax
from jax.experimental import pallas as pl
from jax.experimental.pallas import tpu as pltpu
from jax.experimental.pallas import tpu_sc as plsc

_B, _H, _L = 16, 512, 64
_ROWS = _B * _H
_HB = 256
_NLAYER = 32
_CW = _NLAYER * _L
_NW = 32
_RB = 4
_RPW = _ROWS // _NW
_NCHUNK = _RPW // _RB


def _pattern_np():
    r = np.arange(_L)[:, None]
    c = np.arange(_L)[None, :]
    d = c - r
    pat = (d >= 0) & (d <= 15)
    pat |= (r % 2 == 0) & (d % 2 == 1) & (d >= 17) & (d <= 31)
    pat |= (r % 4 == 0) & (d % 4 == 3) & (d >= 35)
    return pat


def _index_tables():
    pat = _pattern_np()
    rr, cc = np.nonzero(pat)
    d = cc - rr
    layer = np.where(
        d <= 15, d,
        np.where(d <= 31, 16 + (d - 17) // 2, 24 + (d - 35) // 4))
    src = (layer * _L + rr).astype(np.int32)
    return src, rr.astype(np.int32), cc.astype(np.int32), src.size


_ISRC, _IR, _IC, _NIDX = _index_tables()
_NVEC = _NIDX // 16


def _layers_kernel(x_ref, out_ref):
    xb = x_ref[...]
    layers = []
    cur = xb
    layers.append(cur)
    for _ in range(15):
        cur = jnp.maximum(cur, jnp.roll(cur, -1, axis=-1))
        layers.append(cur)
    w16 = cur
    for i in range(8):
        layers.append(jnp.maximum(w16, jnp.roll(w16, -(2 + 2 * i), axis=-1)))
    w32 = jnp.maximum(w16, jnp.roll(w16, -16, axis=-1))
    for i in range(8):
        layers.append(jnp.maximum(w32, jnp.roll(w32, -(4 + 4 * i), axis=-1)))
    out_ref[...] = jnp.concatenate(layers, axis=-1)


def _mask_kernel(out_ref):
    r = lax.broadcasted_iota(jnp.int32, (_L, _L), 0)
    c = lax.broadcasted_iota(jnp.int32, (_L, _L), 1)
    d = c - r
    pat = (d >= 0) & (d <= 15)
    pat |= (r % 2 == 0) & (d % 2 == 1) & (d >= 17) & (d <= 31)
    pat |= (r % 4 == 0) & (d % 4 == 3) & (d >= 35)
    out_ref[...] = jnp.broadcast_to(
        pat.astype(out_ref.dtype)[None, None], out_ref.shape)


def _expand_kernel(c_hbm, zin_hbm, isrc_hbm, ir_hbm, ic_hbm, out_hbm,
                   cin0, cin1, vout0, vout1, isrc_v, ir_v, ic_v,
                   ci_sem0, ci_sem1, vo_sem0, vo_sem1):
    wid = lax.axis_index("s") * 2 + lax.axis_index("c")
    row0 = wid * _RPW
    b = row0 // _H
    h0 = row0 % _H
    cins = (cin0, cin1)
    vouts = (vout0, vout1)
    ci_sems = (ci_sem0, ci_sem1)
    vo_sems = (vo_sem0, vo_sem1)
    pltpu.sync_copy(isrc_hbm, isrc_v)
    pltpu.sync_copy(ir_hbm, ir_v)
    pltpu.sync_copy(ic_hbm, ic_v)
    pltpu.sync_copy(zin_hbm, vout0)
    pltpu.sync_copy(zin_hbm, vout1)

    def cin_dma(ch, t):
        return pltpu.make_async_copy(
            c_hbm.at[pl.ds(row0 + ch * _RB, _RB)], cins[t], ci_sems[t])

    def vout_dma(ch, t):
        return pltpu.make_async_copy(
            vouts[t], out_hbm.at[b, pl.ds(h0 + ch * _RB, _RB)], vo_sems[t])

    cin_dma(0, 0).start()

    def pair_body(g2, _):
        for t in range(2):
            ch = 2 * g2 + t
            cin_dma(ch, t).wait()
            if t == 0:
                cin_dma(ch + 1, 1).start()
            else:
                @pl.when(g2 < _NCHUNK // 2 - 1)
                def _():
                    cin_dma(2 * g2 + 2, 0).start()
            @pl.when(g2 > 0)
            def _():
                vout_dma(ch - 2, t).wait()
            cin_t = cins[t]
            vout_t = vouts[t]

            @plsc.parallel_loop(0, _NVEC, unroll=8)
            def vec_body(v):
                o = v * 16
                isv = isrc_v[pl.ds(o, 16)]
                irv = ir_v[pl.ds(o, 16)]
                icv = ic_v[pl.ds(o, 16)]
                for k in range(_RB):
                    ikv = jnp.full((16,), k, jnp.int32)
                    vals = plsc.load_gather(cin_t, [ikv, isv])
                    plsc.store_scatter(vout_t, [ikv, irv, icv], vals)

            vout_dma(ch, t).start()
        return 0

    lax.fori_loop(0, _NCHUNK // 2, pair_body, 0)
    vout_dma(_NCHUNK - 2, 0).wait()
    vout_dma(_NCHUNK - 1, 1).wait()


def _expand(c_tab):
    mesh = plsc.VectorSubcoreMesh(core_axis_name="c", subcore_axis_name="s")
    kern = pl.kernel(
        _expand_kernel,
        out_type=jax.ShapeDtypeStruct((_B, _H, _L, _L), jnp.float32),
        mesh=mesh,
        compiler_params=pltpu.CompilerParams(needs_layout_passes=False),
        scratch_types=[
            pltpu.VMEM((_RB, _CW), jnp.float32),
            pltpu.VMEM((_RB, _CW), jnp.float32),
            pltpu.VMEM((_RB, _L, _L), jnp.float32),
            pltpu.VMEM((_RB, _L, _L), jnp.float32),
            pltpu.VMEM((_NIDX,), jnp.int32),
            pltpu.VMEM((_NIDX,), jnp.int32),
            pltpu.VMEM((_NIDX,), jnp.int32),
            pltpu.SemaphoreType.DMA,
            pltpu.SemaphoreType.DMA,
            pltpu.SemaphoreType.DMA,
            pltpu.SemaphoreType.DMA,
        ],
    )
    zin = jnp.zeros((_RB, _L, _L), jnp.float32)
    return kern(c_tab, zin, jnp.asarray(_ISRC),
                jnp.asarray(_IR), jnp.asarray(_IC))


@jax.jit
def kernel(x):
    x2 = x.reshape(_ROWS, _L)
    c_tab = pl.pallas_call(
        _layers_kernel,
        grid=(_ROWS // _HB,),
        in_specs=[pl.BlockSpec((_HB, _L), lambda j: (j, 0))],
        out_specs=pl.BlockSpec((_HB, _CW), lambda j: (j, 0)),
        out_shape=jax.ShapeDtypeStruct((_ROWS, _CW), x.dtype),
    )(x2)
    ori_h = _expand(c_tab)
    ori_mask = pl.pallas_call(
        _mask_kernel,
        out_shape=jax.ShapeDtypeStruct((_B, 1, _L, _L), x.dtype),
    )()
    return ori_h, ori_mask


# speedup vs baseline: 2.2686x; 1.0005x over previous
import jax
import jax.numpy as jnp
import numpy as np
from jax import lax
from jax.experimental import pallas as pl
from jax.experimental.pallas import tpu as pltpu
from jax.experimental.pallas import tpu_sc as plsc

_B, _H, _L = 16, 512, 64
_ROWS = _B * _H
_HB = 256
_NLAYER = 32
_CW = _NLAYER * _L
_NW = 32
_RB = 4
_RPW = _ROWS // _NW
_NCHUNK = _RPW // _RB


def _pattern_np():
    r = np.arange(_L)[:, None]
    c = np.arange(_L)[None, :]
    d = c - r
    pat = (d >= 0) & (d <= 15)
    pat |= (r % 2 == 0) & (d % 2 == 1) & (d >= 17) & (d <= 31)
    pat |= (r % 4 == 0) & (d % 4 == 3) & (d >= 35)
    return pat


def _index_tables():
    pat = _pattern_np()
    rr, cc = np.nonzero(pat)
    d = cc - rr
    layer = np.where(
        d <= 15, d,
        np.where(d <= 31, 16 + (d - 17) // 2, 24 + (d - 35) // 4))
    src = (layer * _L + rr).astype(np.int32)
    return src, rr.astype(np.int32), cc.astype(np.int32), src.size


_ISRC, _IR, _IC, _NIDX = _index_tables()
_NVEC = _NIDX // 16


def _layers_kernel(x_ref, out_ref):
    xb = x_ref[...]
    layers = []
    cur = xb
    layers.append(cur)
    for _ in range(15):
        cur = jnp.maximum(cur, jnp.roll(cur, -1, axis=-1))
        layers.append(cur)
    w16 = cur
    for i in range(8):
        layers.append(jnp.maximum(w16, jnp.roll(w16, -(2 + 2 * i), axis=-1)))
    w32 = jnp.maximum(w16, jnp.roll(w16, -16, axis=-1))
    for i in range(8):
        layers.append(jnp.maximum(w32, jnp.roll(w32, -(4 + 4 * i), axis=-1)))
    out_ref[...] = jnp.concatenate(layers, axis=-1)


def _mask_kernel(out_ref):
    r = lax.broadcasted_iota(jnp.int32, (_L, _L), 0)
    c = lax.broadcasted_iota(jnp.int32, (_L, _L), 1)
    d = c - r
    pat = (d >= 0) & (d <= 15)
    pat |= (r % 2 == 0) & (d % 2 == 1) & (d >= 17) & (d <= 31)
    pat |= (r % 4 == 0) & (d % 4 == 3) & (d >= 35)
    out_ref[...] = jnp.broadcast_to(
        pat.astype(out_ref.dtype)[None, None], out_ref.shape)


def _expand_kernel(c_hbm, zin_hbm, isrc_hbm, ir_hbm, ic_hbm, out_hbm,
                   cin0, cin1, vout0, vout1, isrc_v, ir_v, ic_v,
                   ci_sem0, ci_sem1, vo_sem0, vo_sem1):
    wid = lax.axis_index("s") * 2 + lax.axis_index("c")
    row0 = wid * _RPW
    b = row0 // _H
    h0 = row0 % _H
    cins = (cin0, cin1)
    vouts = (vout0, vout1)
    ci_sems = (ci_sem0, ci_sem1)
    vo_sems = (vo_sem0, vo_sem1)
    pltpu.sync_copy(isrc_hbm, isrc_v)
    pltpu.sync_copy(ir_hbm, ir_v)
    pltpu.sync_copy(ic_hbm, ic_v)
    pltpu.sync_copy(zin_hbm, vout0)
    pltpu.sync_copy(zin_hbm, vout1)

    def cin_dma(ch, t):
        return pltpu.make_async_copy(
            c_hbm.at[pl.ds(row0 + ch * _RB, _RB)], cins[t], ci_sems[t])

    def vout_dma(ch, t):
        return pltpu.make_async_copy(
            vouts[t], out_hbm.at[b, pl.ds(h0 + ch * _RB, _RB)], vo_sems[t])

    cin_dma(0, 0).start()

    def pair_body(g2, _):
        for t in range(2):
            ch = 2 * g2 + t
            cin_dma(ch, t).wait()
            if t == 0:
                cin_dma(ch + 1, 1).start()
            else:
                @pl.when(g2 < _NCHUNK // 2 - 1)
                def _():
                    cin_dma(2 * g2 + 2, 0).start()
            @pl.when(g2 > 0)
            def _():
                vout_dma(ch - 2, t).wait()
            cin_t = cins[t]
            vout_t = vouts[t]

            @plsc.parallel_loop(0, _NVEC, unroll=8)
            def vec_body(v):
                o = v * 16
                isv = isrc_v[pl.ds(o, 16)]
                irv = ir_v[pl.ds(o, 16)]
                icv = ic_v[pl.ds(o, 16)]
                for k in range(_RB):
                    ikv = jnp.full((16,), k, jnp.int32)
                    vals = plsc.load_gather(cin_t, [ikv, isv])
                    plsc.store_scatter(vout_t, [ikv, irv, icv], vals)

            vout_dma(ch, t).start()
        return 0

    lax.fori_loop(0, _NCHUNK // 2, pair_body, 0)
    vout_dma(_NCHUNK - 2, 0).wait()
    vout_dma(_NCHUNK - 1, 1).wait()


def _expand(c_tab):
    mesh = plsc.VectorSubcoreMesh(core_axis_name="c", subcore_axis_name="s")
    kern = pl.kernel(
        _expand_kernel,
        out_type=jax.ShapeDtypeStruct((_B, _H, _L, _L), jnp.float32),
        mesh=mesh,
        compiler_params=pltpu.CompilerParams(needs_layout_passes=False),
        scratch_types=[
            pltpu.VMEM((_RB, _CW), jnp.float32),
            pltpu.VMEM((_RB, _CW), jnp.float32),
            pltpu.VMEM((_RB, _L, _L), jnp.float32),
            pltpu.VMEM((_RB, _L, _L), jnp.float32),
            pltpu.VMEM((_NIDX,), jnp.int32),
            pltpu.VMEM((_NIDX,), jnp.int32),
            pltpu.VMEM((_NIDX,), jnp.int32),
            pltpu.SemaphoreType.DMA,
            pltpu.SemaphoreType.DMA,
            pltpu.SemaphoreType.DMA,
            pltpu.SemaphoreType.DMA,
        ],
    )
    zin = jnp.zeros((_RB, _L, _L), jnp.float32)
    return kern(c_tab, zin, jnp.asarray(_ISRC),
                jnp.asarray(_IR), jnp.asarray(_IC))


@jax.jit
def kernel(x):
    x2 = x.reshape(_ROWS, _L)
    c_tab = pl.pallas_call(
        _layers_kernel,
        grid=(_ROWS // _HB,),
        in_specs=[pl.BlockSpec((_HB, _L), lambda j: (j, 0))],
        out_specs=pl.BlockSpec((_HB, _CW), lambda j: (j, 0)),
        out_shape=jax.ShapeDtypeStruct((_ROWS, _CW), x.dtype),
    )(x2)
    ori_h = _expand(c_tab)
    ori_mask = pl.pallas_call(
        _mask_kernel,
        out_shape=jax.ShapeDtypeStruct((_B, 1, _L, _L), x.dtype),
    )()
    return ori_h, ori_mask
